# Initial kernel scaffold; baseline (speedup 1.0000x reference)
import jax
import jax.numpy as jnp
from jax.experimental import pallas as pl


def kernel(h, edge_index, he, mlp0_W1, mlp0_b1, mlp0_W2, mlp0_b2, ln0_g, ln0_b, mlp1_W1, mlp1_b1, mlp1_W2, mlp1_b2, ln1_g, ln1_b, pred0_W, pred0_b, pred1_W, pred1_b, pred2_W, pred2_b):
    raise NotImplementedError("write your pallas kernel here")


# speedup vs baseline: 1.4932x; 1.4932x over previous
import functools

import jax
import jax.numpy as jnp
from jax import lax
from jax.experimental import pallas as pl
from jax.experimental.pallas import tpu as pltpu
from jax.experimental.pallas import tpu_sc as plsc

N = 10000
E = 320000
D = 128

NC = 2
NS = 16
NW = NC * NS
NPW = 320
NPAD = NW * NPW
C = 8000
NCHUNK = E // C
G = 128
MBUF = ((C + G - 1) // G) * G


def _seg_max_body(src_hbm, dst_hbm, h_hbm, out_hbm,
                  src_v, dst_v, msrc, mrow, rows, acc, sem):
    wid = lax.axis_index("s") * NC + lax.axis_index("c")
    lo = wid * NPW

    neg_inf = jnp.full((16,), -jnp.inf, dtype=jnp.float32)
    iota16 = lax.broadcasted_iota(jnp.int32, (16,), 0)

    def init_acc(i, c):
        for f in range(8):
            acc[i, pl.ds(16 * f, 16)] = neg_inf
        return c
    lax.fori_loop(0, NPW, init_acc, 0)

    pad_src = jnp.full((16,), 0, dtype=jnp.int32) + lo

    def init_msrc(i, c):
        msrc[pl.ds(16 * i, 16)] = pad_src
        return c
    lax.fori_loop(0, MBUF // 16, init_msrc, 0)

    def chunk_body(ci, c):
        base = ci * C
        pltpu.sync_copy(src_hbm.at[pl.ds(base, C)], src_v)
        pltpu.sync_copy(dst_hbm.at[pl.ds(base, C)], dst_v)

        def scan_body(i, off):
            d = dst_v[pl.ds(16 * i, 16)]
            s = src_v[pl.ds(16 * i, 16)]
            m = (d >= lo) & (d < lo + NPW)
            inc = plsc.cumsum(jnp.where(m, 1, 0))
            pos = off + inc - 1
            plsc.store_scatter(msrc, [pos], s, mask=m)
            plsc.store_scatter(mrow, [pos], d - lo, mask=m)
            return off + jnp.max(inc)

        mtot = lax.fori_loop(0, C // 16, scan_body, 0)

        mceil = ((mtot + 15) // 16) * 16
        pm = (mtot + iota16) < mceil
        plsc.store_scatter(mrow, [mtot + iota16],
                           jnp.full((16,), NPW, dtype=jnp.int32), mask=pm)
        plsc.store_scatter(msrc, [mtot + iota16], pad_src, mask=pm)

        def blk_body(g, c2):
            pltpu.async_copy(h_hbm.at[msrc.at[pl.ds(g * G, G)]], rows,
                             sem).wait()
            lim = jnp.minimum(mceil - g * G, G)

            def grp(q, c3):
                rv = mrow[pl.ds(g * G + 16 * q, 16)]
                for j in range(16):
                    r = rv[j]
                    b = 16 * q + j
                    for f in range(8):
                        sl = pl.ds(16 * f, 16)
                        acc[r, sl] = jnp.maximum(acc[r, sl], rows[b, sl])
                return c3
            lax.fori_loop(0, lim // 16, grp, 0)
            return c2
        lax.fori_loop(0, (mceil + G - 1) // G, blk_body, 0)
        return c
    lax.fori_loop(0, NCHUNK, chunk_body, 0)

    def fixup(i, c):
        for f in range(8):
            sl = pl.ds(16 * f, 16)
            v = acc[i, sl]
            acc[i, sl] = jnp.where(v == -jnp.inf, 0.0, v)
        return c
    lax.fori_loop(0, NPW, fixup, 0)

    pltpu.sync_copy(acc.at[pl.ds(0, NPW)], out_hbm.at[pl.ds(lo, NPW)])


@functools.cache
def _seg_max():
    return pl.kernel(
        _seg_max_body,
        out_type=jax.ShapeDtypeStruct((NPAD, D), jnp.float32),
        mesh=plsc.VectorSubcoreMesh(
            core_axis_name="c", subcore_axis_name="s", num_cores=NC,
            num_subcores=NS),
        compiler_params=pltpu.CompilerParams(needs_layout_passes=False),
        scratch_types=[
            pltpu.VMEM((C,), jnp.int32),
            pltpu.VMEM((C,), jnp.int32),
            pltpu.VMEM((MBUF,), jnp.int32),
            pltpu.VMEM((MBUF,), jnp.int32),
            pltpu.VMEM((G, D), jnp.float32),
            pltpu.VMEM((NPW + 1, D), jnp.float32),
            pltpu.SemaphoreType.DMA,
        ],
    )


def _layer_body(want_sum_in, h_ref, agg_ref, W1_ref, b1_ref, W2_ref, b2_ref,
                g_ref, b_ref, *rest):
    if want_sum_in:
        y_ref, sum_ref, sumin_ref = rest
    else:
        y_ref, sum_ref = rest
        sumin_ref = None
    i = pl.program_id(0)
    hb = h_ref[...]
    x = hb + agg_ref[...]
    t = jnp.maximum(
        jnp.dot(x, W1_ref[...], preferred_element_type=jnp.float32,
                precision=lax.Precision.HIGHEST) + b1_ref[...], 0.0)
    y = jnp.dot(t, W2_ref[...], preferred_element_type=jnp.float32,
                precision=lax.Precision.HIGHEST) + b2_ref[...]
    mu = jnp.mean(y, axis=1, keepdims=True)
    var = jnp.mean((y - mu) ** 2, axis=1, keepdims=True)
    out = jnp.maximum((y - mu) / jnp.sqrt(var + 1e-5) * g_ref[...]
                      + b_ref[...], 0.0)
    y_ref[...] = out

    @pl.when(i == 0)
    def _():
        sum_ref[...] = jnp.zeros_like(sum_ref)
        if want_sum_in:
            sumin_ref[...] = jnp.zeros_like(sumin_ref)

    sum_ref[...] += jnp.sum(out, axis=0, keepdims=True)
    if want_sum_in:
        sumin_ref[...] += jnp.sum(hb, axis=0, keepdims=True)


_BLK = 1000


def _make_layer(want_sum_in):
    n_out = 3 if want_sum_in else 2
    vec = pl.BlockSpec((1, D), lambda i: (0, 0))
    mat = pl.BlockSpec((D, D), lambda i: (0, 0))
    row = pl.BlockSpec((_BLK, D), lambda i: (i, 0))
    out_shapes = [jax.ShapeDtypeStruct((N, D), jnp.float32)] + \
        [jax.ShapeDtypeStruct((1, D), jnp.float32)] * (n_out - 1)
    out_specs = [row] + [vec] * (n_out - 1)
    return pl.pallas_call(
        functools.partial(_layer_body, want_sum_in),
        grid=(N // _BLK,),
        in_specs=[row, row, mat, vec, mat, vec, vec, vec],
        out_specs=out_specs,
        out_shape=out_shapes,
        compiler_params=pltpu.CompilerParams(
            dimension_semantics=("arbitrary",)),
    )


_layer0 = _make_layer(True)
_layer1 = _make_layer(False)


def _score_body(sh_ref, s1_ref, s2_ref, W0_ref, b0_ref, W1_ref, b1_ref,
                W2_ref, b2_ref, out_ref):
    inv_n = jnp.float32(1.0 / N)
    acc = jnp.dot(sh_ref[...] * inv_n, W0_ref[...],
                  preferred_element_type=jnp.float32,
                  precision=lax.Precision.HIGHEST) + b0_ref[...]
    acc += jnp.dot(s1_ref[...] * inv_n, W1_ref[...],
                   preferred_element_type=jnp.float32,
                   precision=lax.Precision.HIGHEST) + b1_ref[...]
    acc += jnp.dot(s2_ref[...] * inv_n, W2_ref[...],
                   preferred_element_type=jnp.float32,
                   precision=lax.Precision.HIGHEST) + b2_ref[...]
    out_ref[...] = acc


_score = pl.pallas_call(
    _score_body,
    out_shape=jax.ShapeDtypeStruct((1, D), jnp.float32),
)


def kernel(h, edge_index, he, mlp0_W1, mlp0_b1, mlp0_W2, mlp0_b2, ln0_g,
           ln0_b, mlp1_W1, mlp1_b1, mlp1_W2, mlp1_b2, ln1_g, ln1_b, pred0_W,
           pred0_b, pred1_W, pred1_b, pred2_W, pred2_b):
    del he
    r = lambda v: v.reshape(1, D)

    seg_max = _seg_max()
    src = edge_index[0]
    dst = edge_index[1]
    agg0 = seg_max(src, dst, h)[:N]
    x1, sum1, sumh = _layer0(h, agg0, mlp0_W1, r(mlp0_b1), mlp0_W2,
                             r(mlp0_b2), r(ln0_g), r(ln0_b))
    agg1 = seg_max(src, dst, x1)[:N]
    x2, sum2 = _layer1(x1, agg1, mlp1_W1, r(mlp1_b1), mlp1_W2, r(mlp1_b2),
                       r(ln1_g), r(ln1_b))
    score = _score(sumh, sum1, sum2, pred0_W, r(pred0_b), pred1_W,
                   r(pred1_b), pred2_W, r(pred2_b))
    return (x2, score)
